# baseline (speedup 1.0000x reference)
import jax
import jax.numpy as jnp
from jax.experimental import pallas as pl

_BIG = 3.0e38
_B = 1024
_K = 1024
_D = 256


def _split2(v):
    hi = v.astype(jnp.bfloat16)
    lo = (v - hi.astype(jnp.float32)).astype(jnp.bfloat16)
    return hi, lo


def _kmeans_assign_kernel(x_ref, c_ref, o_ref):
    x = x_ref[:]
    c = c_ref[:]

    x_hi, x_lo = _split2(x)
    c_hi, c_lo = _split2(c)
    c_lo2 = (c - c_hi.astype(jnp.float32) - c_lo.astype(jnp.float32)
             ).astype(jnp.bfloat16)

    ncn = -0.5 * jnp.sum(c * c, axis=1, keepdims=True)
    ncn_hi, ncn_lo = _split2(ncn)
    ncn_lo2 = (ncn - ncn_hi.astype(jnp.float32) - ncn_lo.astype(jnp.float32)
               ).astype(jnp.bfloat16)

    ones3 = jnp.ones((_B, 3), dtype=jnp.bfloat16)
    xcat = jnp.concatenate([x_hi, x_hi, x_lo, ones3], axis=1)
    ccat = jnp.concatenate(
        [c_hi, c_lo, c_hi, ncn_hi, ncn_lo, ncn_lo2], axis=1)

    score = jax.lax.dot_general(
        xcat, ccat, (((1,), (1,)), ((), ())),
        preferred_element_type=jnp.float32)

    m1 = jnp.max(score, axis=1, keepdims=True)
    eq1 = score == m1
    oh1 = eq1.astype(jnp.bfloat16)
    masked = jnp.where(eq1, -_BIG, score)
    m2 = jnp.max(masked, axis=1, keepdims=True)
    eq2 = masked == m2
    oh2 = eq2.astype(jnp.bfloat16)

    coli = jax.lax.broadcasted_iota(jnp.int32, (_K, 1), 0)
    colv = coli.astype(jnp.float32)
    col_hi = ((coli - (coli % 8)).astype(jnp.float32)).astype(jnp.bfloat16)
    col_lo = (colv - col_hi.astype(jnp.float32)).astype(jnp.bfloat16)
    onesc = jnp.ones((_K, 1), dtype=jnp.bfloat16)
    chat = jnp.concatenate(
        [c_hi, c_lo, c_lo2, col_hi, col_lo, onesc], axis=1)

    def _gather(oh):
        g = jax.lax.dot_general(oh, chat, (((1,), (0,)), ((), ())),
                                preferred_element_type=jnp.float32)
        crow = g[:, :_D] + g[:, _D:2 * _D] + g[:, 2 * _D:3 * _D]
        idx = (g[:, 3 * _D:3 * _D + 1] + g[:, 3 * _D + 1:3 * _D + 2]
               ).astype(jnp.int32)
        cnt = g[:, 3 * _D + 2:3 * _D + 3]
        return crow, idx, cnt

    c1, i1, n1 = _gather(oh1)
    c2, i2, n2 = _gather(oh2)
    d1 = jnp.sum(jnp.square(x - c1), axis=1, keepdims=True)
    d2 = jnp.sum(jnp.square(x - c2), axis=1, keepdims=True)

    o_ref[:] = jnp.where(
        d1 < d2, i1, jnp.where(d2 < d1, i2, jnp.minimum(i1, i2))
    ).astype(jnp.int32)

    ties = jnp.max(jnp.maximum(n1, n2)) > 1.5

    @pl.when(ties)
    def _slow_path():
        col = jax.lax.broadcasted_iota(jnp.int32, (_B, _K), 1)
        si1 = jnp.min(jnp.where(score == m1, col, _K), axis=1, keepdims=True)
        smasked = jnp.where(col == si1, -_BIG, score)
        sm2 = jnp.max(smasked, axis=1, keepdims=True)
        si2 = jnp.min(jnp.where(smasked == sm2, col, _K), axis=1,
                      keepdims=True)
        soh1 = (col == si1).astype(jnp.bfloat16)
        soh2 = (col == si2).astype(jnp.bfloat16)
        sc1, _, _ = _gather(soh1)
        sc2, _, _ = _gather(soh2)
        sd1 = jnp.sum(jnp.square(x - sc1), axis=1, keepdims=True)
        sd2 = jnp.sum(jnp.square(x - sc2), axis=1, keepdims=True)
        o_ref[:] = jnp.where(
            sd1 < sd2, si1, jnp.where(sd2 < sd1, si2, jnp.minimum(si1, si2))
        ).astype(jnp.int32)


def kernel(points, centroids):
    out = pl.pallas_call(
        _kmeans_assign_kernel,
        out_shape=jax.ShapeDtypeStruct((_B, 1), jnp.int32),
    )(points, centroids)
    return out[:, 0]


# speedup vs baseline: 1.0697x; 1.0697x over previous
import jax
import jax.numpy as jnp
from jax.experimental import pallas as pl

_BIG = 3.0e38
_B = 1024
_K = 1024
_D = 256


def _split2(v):
    hi = v.astype(jnp.bfloat16)
    lo = (v - hi.astype(jnp.float32)).astype(jnp.bfloat16)
    return hi, lo


def _kmeans_assign_kernel(x_ref, c_ref, o_ref):
    x = x_ref[:]
    c = c_ref[:]

    x_hi, x_lo = _split2(x)
    c_hi, c_lo = _split2(c)
    c_lo2 = (c - c_hi.astype(jnp.float32) - c_lo.astype(jnp.float32)
             ).astype(jnp.bfloat16)

    ncn = -0.5 * jnp.sum(c * c, axis=1, keepdims=True)
    ncn_hi, ncn_lo = _split2(ncn)
    ncn_lo2 = (ncn - ncn_hi.astype(jnp.float32) - ncn_lo.astype(jnp.float32)
               ).astype(jnp.bfloat16)

    ones3 = jnp.ones((_B, 3), dtype=jnp.bfloat16)
    xcat = jnp.concatenate([x_hi, x_hi, x_lo, ones3], axis=1)
    ccat = jnp.concatenate(
        [c_hi, c_lo, c_hi, ncn_hi, ncn_lo, ncn_lo2], axis=1)

    score = jax.lax.dot_general(
        xcat, ccat, (((1,), (1,)), ((), ())),
        preferred_element_type=jnp.float32)

    col = jax.lax.broadcasted_iota(jnp.int32, (_B, _K), 1)

    m1 = jnp.max(score, axis=1, keepdims=True)
    i1 = jnp.min(jnp.where(score == m1, col, _K), axis=1, keepdims=True)

    masked = jnp.where(col == i1, -_BIG, score)
    m2 = jnp.max(masked, axis=1, keepdims=True)
    i2 = jnp.min(jnp.where(masked == m2, col, _K), axis=1, keepdims=True)

    oh1 = (col == i1).astype(jnp.bfloat16)
    oh2 = (col == i2).astype(jnp.bfloat16)
    chat = jnp.concatenate([c_hi, c_lo, c_lo2], axis=1)

    def _gather(oh):
        g = jax.lax.dot_general(oh, chat, (((1,), (0,)), ((), ())),
                                preferred_element_type=jnp.float32)
        return g[:, :_D] + g[:, _D:2 * _D] + g[:, 2 * _D:]

    c1 = _gather(oh1)
    c2 = _gather(oh2)
    d1 = jnp.sum(jnp.square(x - c1), axis=1, keepdims=True)
    d2 = jnp.sum(jnp.square(x - c2), axis=1, keepdims=True)

    o_ref[:] = jnp.where(
        d1 < d2, i1, jnp.where(d2 < d1, i2, jnp.minimum(i1, i2))
    ).astype(jnp.int32)


def kernel(points, centroids):
    out = pl.pallas_call(
        _kmeans_assign_kernel,
        out_shape=jax.ShapeDtypeStruct((_B, 1), jnp.int32),
    )(points, centroids)
    return out[:, 0]


# speedup vs baseline: 1.0931x; 1.0218x over previous
import jax
import jax.numpy as jnp
from jax.experimental import pallas as pl

_BIG = 3.0e38
_B = 1024
_K = 1024
_D = 256


def _split2(v):
    hi = v.astype(jnp.bfloat16)
    lo = (v - hi.astype(jnp.float32)).astype(jnp.bfloat16)
    return hi, lo


def _kmeans_assign_kernel(x_ref, c_ref, o_ref):
    x = x_ref[:]
    c = c_ref[:]

    x_hi, x_lo = _split2(x)
    c_hi, c_lo = _split2(c)
    c_lo2 = (c - c_hi.astype(jnp.float32) - c_lo.astype(jnp.float32)
             ).astype(jnp.bfloat16)

    ncn = -0.5 * jnp.sum(c * c, axis=1, keepdims=True)
    ncn_hi, ncn_lo = _split2(ncn)
    ncn_lo2 = (ncn - ncn_hi.astype(jnp.float32) - ncn_lo.astype(jnp.float32)
               ).astype(jnp.bfloat16)

    ones3 = jnp.ones((_B, 3), dtype=jnp.bfloat16)
    xcat = jnp.concatenate([x_hi, x_hi, x_lo, ones3], axis=1)
    ccat = jnp.concatenate(
        [c_hi, c_lo, c_hi, ncn_hi, ncn_lo, ncn_lo2], axis=1)

    score = jax.lax.dot_general(
        xcat, ccat, (((1,), (1,)), ((), ())),
        preferred_element_type=jnp.float32)

    col = jax.lax.broadcasted_iota(jnp.int32, (_B, _K), 1)

    m1 = jnp.max(score, axis=1, keepdims=True)
    i1 = jnp.min(jnp.where(score == m1, col, _K), axis=1, keepdims=True)

    mask1 = col == i1
    oh1 = mask1.astype(jnp.bfloat16)
    masked = jnp.where(mask1, -_BIG, score)
    m2 = jnp.max(masked, axis=1, keepdims=True)
    eq2 = masked == m2
    i2 = jnp.min(jnp.where(eq2, col, _K), axis=1, keepdims=True)
    oh2 = eq2.astype(jnp.bfloat16)

    chat = jnp.concatenate([c_hi, c_lo, c_lo2], axis=1)
    ohs = jnp.concatenate([oh1, oh2], axis=0)
    g = jax.lax.dot_general(ohs, chat, (((1,), (0,)), ((), ())),
                            preferred_element_type=jnp.float32)
    gsum = g[:, :_D] + g[:, _D:2 * _D] + g[:, 2 * _D:]
    c1 = gsum[:_B]
    c2 = gsum[_B:]
    d1 = jnp.sum(jnp.square(x - c1), axis=1, keepdims=True)
    d2 = jnp.sum(jnp.square(x - c2), axis=1, keepdims=True)

    o_ref[:] = jnp.where(
        d1 < d2, i1, jnp.where(d2 < d1, i2, jnp.minimum(i1, i2))
    ).astype(jnp.int32)


def kernel(points, centroids):
    out = pl.pallas_call(
        _kmeans_assign_kernel,
        out_shape=jax.ShapeDtypeStruct((_B, 1), jnp.int32),
    )(points, centroids)
    return out[:, 0]


# speedup vs baseline: 1.1064x; 1.0122x over previous
import jax
import jax.numpy as jnp
from jax.experimental import pallas as pl

_BIG = 3.0e38
_B = 1024
_K = 1024
_D = 256


def _split2(v):
    hi = v.astype(jnp.bfloat16)
    lo = (v - hi.astype(jnp.float32)).astype(jnp.bfloat16)
    return hi, lo


def _kmeans_assign_kernel(x_ref, c_ref, o_ref):
    x = x_ref[:]
    c = c_ref[:]

    x_hi, x_lo = _split2(x)
    c_hi, c_lo = _split2(c)
    c_lo2 = (c - c_hi.astype(jnp.float32) - c_lo.astype(jnp.float32)
             ).astype(jnp.bfloat16)

    ncn = -0.5 * jnp.sum(c * c, axis=1, keepdims=True)
    ncn_hi, ncn_lo = _split2(ncn)
    ncn_lo2 = (ncn - ncn_hi.astype(jnp.float32) - ncn_lo.astype(jnp.float32)
               ).astype(jnp.bfloat16)

    ones3 = jnp.ones((_B, 3), dtype=jnp.bfloat16)
    xcat = jnp.concatenate([x_hi, x_hi, x_lo, ones3], axis=1)
    ccat = jnp.concatenate(
        [c_hi, c_lo, c_hi, ncn_hi, ncn_lo, ncn_lo2], axis=1)

    score = jax.lax.dot_general(
        xcat, ccat, (((1,), (1,)), ((), ())),
        preferred_element_type=jnp.float32)

    col = jax.lax.broadcasted_iota(jnp.int32, (_B, _K), 1)

    m1 = jnp.max(score, axis=1, keepdims=True)
    i1 = jnp.min(jnp.where(score == m1, col, _K), axis=1, keepdims=True)

    mask1 = col == i1
    oh1 = mask1.astype(jnp.bfloat16)
    masked = jnp.where(mask1, -_BIG, score)
    m2 = jnp.max(masked, axis=1, keepdims=True)
    eq2 = masked == m2
    i2 = jnp.min(jnp.where(eq2, col, _K), axis=1, keepdims=True)
    oh2 = eq2.astype(jnp.bfloat16)

    chat = jnp.concatenate([c_hi, c_lo, c_lo2], axis=1)

    def _gather(oh):
        g = jax.lax.dot_general(oh, chat, (((1,), (0,)), ((), ())),
                                preferred_element_type=jnp.float32)
        return g[:, :_D] + g[:, _D:2 * _D] + g[:, 2 * _D:]

    c1 = _gather(oh1)
    c2 = _gather(oh2)
    d1 = jnp.sum(jnp.square(x - c1), axis=1, keepdims=True)
    d2 = jnp.sum(jnp.square(x - c2), axis=1, keepdims=True)

    o_ref[:] = jnp.where(
        d1 < d2, i1, jnp.where(d2 < d1, i2, jnp.minimum(i1, i2))
    ).astype(jnp.int32)


def kernel(points, centroids):
    out = pl.pallas_call(
        _kmeans_assign_kernel,
        out_shape=jax.ShapeDtypeStruct((_B, 1), jnp.int32),
    )(points, centroids)
    return out[:, 0]


# speedup vs baseline: 1.1351x; 1.0259x over previous
import jax
import jax.numpy as jnp
from jax.experimental import pallas as pl

_BIG = 3.0e38
_B = 1024
_K = 1024
_D = 256


def _split2(v):
    hi = v.astype(jnp.bfloat16)
    lo = (v - hi.astype(jnp.float32)).astype(jnp.bfloat16)
    return hi, lo


def _kmeans_assign_kernel(x_ref, c_ref, o_ref):
    x = x_ref[:]
    c = c_ref[:]

    x_hi, x_lo = _split2(x)
    c_hi, c_lo = _split2(c)
    c_lo2 = (c - c_hi.astype(jnp.float32) - c_lo.astype(jnp.float32)
             ).astype(jnp.bfloat16)

    ncn = -0.5 * jnp.sum(c * c, axis=1, keepdims=True)
    ncn_hi, ncn_lo = _split2(ncn)
    ncn_lo2 = (ncn - ncn_hi.astype(jnp.float32) - ncn_lo.astype(jnp.float32)
               ).astype(jnp.bfloat16)

    ones3 = jnp.ones((_B, 3), dtype=jnp.bfloat16)
    xcat = jnp.concatenate([x_hi, x_hi, x_lo, ones3], axis=1)
    ccat = jnp.concatenate(
        [c_hi, c_lo, c_hi, ncn_hi, ncn_lo, ncn_lo2], axis=1)

    score = jax.lax.dot_general(
        xcat, ccat, (((1,), (1,)), ((), ())),
        preferred_element_type=jnp.float32)

    col = jax.lax.broadcasted_iota(jnp.int32, (_B, _K), 1).astype(jnp.float32)
    kf = float(_K)

    m1 = jnp.max(score, axis=1, keepdims=True)
    i1 = jnp.min(jnp.where(score == m1, col, kf), axis=1, keepdims=True)

    mask1 = col == i1
    oh1 = mask1.astype(jnp.bfloat16)
    masked = jnp.where(mask1, -_BIG, score)
    m2 = jnp.max(masked, axis=1, keepdims=True)
    eq2 = masked == m2
    i2 = jnp.min(jnp.where(eq2, col, kf), axis=1, keepdims=True)
    oh2 = eq2.astype(jnp.bfloat16)

    chat = jnp.concatenate([c_hi, c_lo, c_lo2], axis=1)

    def _gather(oh):
        g = jax.lax.dot_general(oh, chat, (((1,), (0,)), ((), ())),
                                preferred_element_type=jnp.float32)
        return g[:, :_D] + g[:, _D:2 * _D] + g[:, 2 * _D:]

    c1 = _gather(oh1)
    c2 = _gather(oh2)
    d1 = jnp.sum(jnp.square(x - c1), axis=1, keepdims=True)
    d2 = jnp.sum(jnp.square(x - c2), axis=1, keepdims=True)

    o_ref[:] = jnp.where(
        d1 < d2, i1, jnp.where(d2 < d1, i2, jnp.minimum(i1, i2))
    ).astype(jnp.int32)


def kernel(points, centroids):
    out = pl.pallas_call(
        _kmeans_assign_kernel,
        out_shape=jax.ShapeDtypeStruct((_B, 1), jnp.int32),
    )(points, centroids)
    return out[:, 0]
